# baseline (speedup 1.0000x reference)
import functools

import jax
import jax.numpy as jnp
from jax import lax
from jax.experimental import pallas as pl
from jax.experimental.pallas import tpu as pltpu
from jax.experimental.pallas import tpu_sc as plsc

NC = 2
NS = 16
NW = NC * NS
CHUNK = 128
NBUF = 2
GROUP = 8
ROW_BLOCK = 1000


def _mm_bias_body(x_ref, w_ref, b_ref, o_ref):
    o_ref[...] = (
        jnp.dot(x_ref[...], w_ref[...], preferred_element_type=jnp.float32)
        + b_ref[...]
    )


def _mm_bias(x, W, b):
    n, d_in = x.shape
    d_out = W.shape[1]
    grid = n // ROW_BLOCK
    return pl.pallas_call(
        _mm_bias_body,
        grid=(grid,),
        in_specs=[
            pl.BlockSpec((ROW_BLOCK, d_in), lambda i: (i, 0)),
            pl.BlockSpec((d_in, d_out), lambda i: (0, 0)),
            pl.BlockSpec((1, d_out), lambda i: (0, 0)),
        ],
        out_specs=pl.BlockSpec((ROW_BLOCK, d_out), lambda i: (i, 0)),
        out_shape=jax.ShapeDtypeStruct((n, d_out), jnp.float32),
    )(x, W, b.reshape(1, d_out))


def _combine_relu_mm_body(acc_ref, w_ref, b_ref, o_ref):
    x1 = jnp.maximum(acc_ref[0] + acc_ref[1], 0.0)
    o_ref[...] = (
        jnp.dot(x1, w_ref[...], preferred_element_type=jnp.float32) + b_ref[...]
    )


def _combine_relu_mm(parts, W, b, n):
    d_in = parts.shape[2]
    d_out = W.shape[1]
    grid = n // ROW_BLOCK
    return pl.pallas_call(
        _combine_relu_mm_body,
        grid=(grid,),
        in_specs=[
            pl.BlockSpec((2, ROW_BLOCK, d_in), lambda i: (0, i, 0)),
            pl.BlockSpec((d_in, d_out), lambda i: (0, 0)),
            pl.BlockSpec((1, d_out), lambda i: (0, 0)),
        ],
        out_specs=pl.BlockSpec((ROW_BLOCK, d_out), lambda i: (i, 0)),
        out_shape=jax.ShapeDtypeStruct((n, d_out), jnp.float32),
    )(parts, W, b.reshape(1, d_out))


def _combine_body(acc_ref, o_ref):
    o_ref[...] = acc_ref[0] + acc_ref[1]


def _combine(parts, n):
    d = parts.shape[2]
    grid = n // ROW_BLOCK
    return pl.pallas_call(
        _combine_body,
        grid=(grid,),
        in_specs=[pl.BlockSpec((2, ROW_BLOCK, d), lambda i: (0, i, 0))],
        out_specs=pl.BlockSpec((ROW_BLOCK, d), lambda i: (i, 0)),
        out_shape=jax.ShapeDtypeStruct((n, d), jnp.float32),
    )(parts)


def _acc_rows(n):
    return ((n + 1 + NS * 8 - 1) // (NS * 8)) * (NS * 8)


def _make_sc_scatter(n, d, ch_per_tile):
    acc_rows = _acc_rows(n)
    per_tile = acc_rows // NS
    assert ch_per_tile % GROUP == 0
    ngroups = ch_per_tile // GROUP
    nch = ch_per_tile
    mesh = plsc.VectorSubcoreMesh(core_axis_name="c", subcore_axis_name="s")

    @functools.partial(
        pl.kernel,
        out_type=jax.ShapeDtypeStruct((NC, acc_rows, d), jnp.float32),
        mesh=mesh,
        scratch_types=[
            pltpu.VMEM((ch_per_tile, CHUNK), jnp.int32),
            pltpu.VMEM((2, GROUP, CHUNK), jnp.int32),
            pltpu.VMEM((CHUNK, d), jnp.float32),
            pltpu.VMEM((CHUNK, d), jnp.float32),
            pltpu.VMEM_SHARED((acc_rows, d), jnp.float32),
            pltpu.SemaphoreType.DMA,
            pltpu.SemaphoreType.DMA,
            pltpu.SemaphoreType.DMA,
            pltpu.SemaphoreType.DMA,
            pltpu.SemaphoreType.DMA,
        ],
    )
    def sc_scatter(support_hbm, src_hbm, dst_hbm, out_hbm,
                   srcp, dstb, rb0, rb1, acc_sh, g0, g1, s0, s1, isem):
        c = lax.axis_index("c")
        s = lax.axis_index("s")
        t = c * NS + s
        bufs = (rb0, rb1)
        gsem = (g0, g1)
        ssem = (s0, s1)

        def drain(sem, buf):
            pltpu.make_async_copy(
                support_hbm.at[pl.ds(0, CHUNK)], buf, sem).wait()

        def zbody(i, _):
            r = i // (d // 16)
            col = (i % (d // 16)) * 16
            rb0[r, pl.ds(col, 16)] = jnp.zeros((16,), jnp.float32)
            return ()
        lax.fori_loop(0, CHUNK * (d // 16), zbody, ())
        zoffs = [(k * CHUNK, min(CHUNK, per_tile - k * CHUNK))
                 for k in range(-(-per_tile // CHUNK))]
        zd = [
            pltpu.async_copy(
                rb0.at[pl.ds(0, nr)],
                acc_sh.at[pl.ds(s * per_tile + r0, nr)], g0)
            for r0, nr in zoffs
        ]
        pltpu.sync_copy(src_hbm.at[t], srcp)
        pltpu.sync_copy(dst_hbm.at[t, pl.ds(0, GROUP)], dstb.at[0])
        for dsc in zd:
            dsc.wait()
        plsc.subcore_barrier()

        pltpu.async_copy(support_hbm.at[srcp.at[0]], bufs[0], gsem[0])

        def group(gi, _):
            b = gi % 2
            nb = 1 - b
            base = gi * GROUP

            @pl.when(gi + 1 < ngroups)
            def _prefetch():
                off = (gi + 1) * GROUP
                pltpu.async_copy(
                    dst_hbm.at[t, pl.ds(off, GROUP)], dstb.at[nb], isem)

            for jj in range(GROUP):
                r = jj % NBUF
                pr = (jj - 1) % NBUF
                m = base + jj
                drain(gsem[r], bufs[r])
                pltpu.async_copy(
                    bufs[r], acc_sh.at[dstb.at[b, jj]], ssem[r], add=True)
                if jj == 0:
                    @pl.when(gi > 0)
                    def _d0():
                        drain(ssem[pr], bufs[pr])
                else:
                    drain(ssem[pr], bufs[pr])
                if jj + 1 < GROUP:
                    pltpu.async_copy(
                        support_hbm.at[srcp.at[m + 1]], bufs[pr], gsem[pr])
                else:
                    @pl.when(gi + 1 < ngroups)
                    def _la():
                        pltpu.async_copy(
                            support_hbm.at[srcp.at[m + 1]], bufs[pr], gsem[pr])
            @pl.when(gi + 1 < ngroups)
            def _drain_prefetch():
                pltpu.make_async_copy(
                    dst_hbm.at[t, pl.ds(0, GROUP)], dstb.at[nb], isem).wait()
            return ()
        lax.fori_loop(0, ngroups, group, ())
        drain(ssem[(nch - 1) % NBUF], bufs[(nch - 1) % NBUF])
        plsc.subcore_barrier()

        pend = [None, None]
        for k, (r0, nr) in enumerate(zoffs):
            p = k & 1
            base = s * per_tile + r0
            if pend[p] is not None:
                pend[p].wait()
            pltpu.sync_copy(acc_sh.at[pl.ds(base, nr)], bufs[p].at[pl.ds(0, nr)])
            pend[p] = pltpu.async_copy(
                bufs[p].at[pl.ds(0, nr)], out_hbm.at[c].at[pl.ds(base, nr)],
                gsem[p])
        for p in (0, 1):
            if pend[p] is not None:
                pend[p].wait()

    return sc_scatter


def kernel(x, adj, W1, b1, W2, b2):
    n, d = x.shape
    e = adj.shape[1]
    ch_per_tile = -(-e // (NW * CHUNK))
    ch_per_tile = ((ch_per_tile + GROUP - 1) // GROUP) * GROUP
    e_pad = NW * ch_per_tile * CHUNK

    src = adj[0].astype(jnp.int32)
    dst = adj[1].astype(jnp.int32)
    pad = e_pad - e
    if pad:
        src = jnp.concatenate([src, jnp.zeros((pad,), jnp.int32)])
        dst = jnp.concatenate([dst, jnp.full((pad,), n, jnp.int32)])
    src_t = src.reshape(NW, ch_per_tile, CHUNK)
    dst_t = dst.reshape(NW, ch_per_tile, CHUNK)

    sc_scatter = _make_sc_scatter(n, d, ch_per_tile)

    support1 = _mm_bias(x, W1, b1)
    parts1 = sc_scatter(support1, src_t, dst_t)
    support2 = _combine_relu_mm(parts1, W2, b2, n)
    parts2 = sc_scatter(support2, src_t, dst_t)
    return _combine(parts2, n)


# speedup vs baseline: 1.0153x; 1.0153x over previous
import functools

import jax
import jax.numpy as jnp
from jax import lax
from jax.experimental import pallas as pl
from jax.experimental.pallas import tpu as pltpu
from jax.experimental.pallas import tpu_sc as plsc

NC = 2
NS = 16
NW = NC * NS
CHUNK = 128
NBUF = 2
GROUP = 8
ROW_BLOCK = 1000


def _mm_bias_body(x_ref, w_ref, b_ref, o_ref):
    o_ref[...] = (
        jnp.dot(x_ref[...], w_ref[...], preferred_element_type=jnp.float32)
        + b_ref[...]
    )


def _mm_bias(x, W, b):
    n, d_in = x.shape
    d_out = W.shape[1]
    grid = n // ROW_BLOCK
    return pl.pallas_call(
        _mm_bias_body,
        grid=(grid,),
        in_specs=[
            pl.BlockSpec((ROW_BLOCK, d_in), lambda i: (i, 0)),
            pl.BlockSpec((d_in, d_out), lambda i: (0, 0)),
            pl.BlockSpec((1, d_out), lambda i: (0, 0)),
        ],
        out_specs=pl.BlockSpec((ROW_BLOCK, d_out), lambda i: (i, 0)),
        out_shape=jax.ShapeDtypeStruct((n, d_out), jnp.float32),
    )(x, W, b.reshape(1, d_out))


def _combine_relu_mm_body(acc_ref, w_ref, b_ref, o_ref):
    x1 = jnp.maximum(acc_ref[0] + acc_ref[1], 0.0)
    o_ref[...] = (
        jnp.dot(x1, w_ref[...], preferred_element_type=jnp.float32) + b_ref[...]
    )


def _combine_relu_mm(parts, W, b, n):
    d_in = parts.shape[2]
    d_out = W.shape[1]
    grid = n // ROW_BLOCK
    return pl.pallas_call(
        _combine_relu_mm_body,
        grid=(grid,),
        in_specs=[
            pl.BlockSpec((2, ROW_BLOCK, d_in), lambda i: (0, i, 0)),
            pl.BlockSpec((d_in, d_out), lambda i: (0, 0)),
            pl.BlockSpec((1, d_out), lambda i: (0, 0)),
        ],
        out_specs=pl.BlockSpec((ROW_BLOCK, d_out), lambda i: (i, 0)),
        out_shape=jax.ShapeDtypeStruct((n, d_out), jnp.float32),
    )(parts, W, b.reshape(1, d_out))


def _combine_body(acc_ref, o_ref):
    o_ref[...] = acc_ref[0] + acc_ref[1]


def _combine(parts, n):
    d = parts.shape[2]
    grid = n // ROW_BLOCK
    return pl.pallas_call(
        _combine_body,
        grid=(grid,),
        in_specs=[pl.BlockSpec((2, ROW_BLOCK, d), lambda i: (0, i, 0))],
        out_specs=pl.BlockSpec((ROW_BLOCK, d), lambda i: (i, 0)),
        out_shape=jax.ShapeDtypeStruct((n, d), jnp.float32),
    )(parts)


def _acc_rows(n):
    return ((n + 1 + NS * 8 - 1) // (NS * 8)) * (NS * 8)


def _make_sc_scatter(n, d, ch_per_tile):
    acc_rows = _acc_rows(n)
    per_tile = acc_rows // NS
    assert ch_per_tile % GROUP == 0
    ngroups = ch_per_tile // GROUP
    nch = ch_per_tile
    mesh = plsc.VectorSubcoreMesh(core_axis_name="c", subcore_axis_name="s")

    @functools.partial(
        pl.kernel,
        out_type=jax.ShapeDtypeStruct((NC, acc_rows, d), jnp.float32),
        mesh=mesh,
        scratch_types=[
            pltpu.VMEM((ch_per_tile, CHUNK), jnp.int32),
            pltpu.VMEM((2, GROUP, CHUNK), jnp.int32),
            pltpu.VMEM((CHUNK, d), jnp.float32),
            pltpu.VMEM((CHUNK, d), jnp.float32),
            pltpu.VMEM_SHARED((acc_rows, d), jnp.float32),
            pltpu.SemaphoreType.DMA,
            pltpu.SemaphoreType.DMA,
            pltpu.SemaphoreType.DMA,
            pltpu.SemaphoreType.DMA,
            pltpu.SemaphoreType.DMA,
        ],
    )
    def sc_scatter(support_hbm, src_hbm, dst_hbm, out_hbm,
                   srcp, dstb, rb0, rb1, acc_sh, g0, g1, s0, s1, isem):
        c = lax.axis_index("c")
        s = lax.axis_index("s")
        t = c * NS + s
        bufs = (rb0, rb1)
        gsem = (g0, g1)
        ssem = (s0, s1)

        def drain(sem, buf):
            pltpu.make_async_copy(
                support_hbm.at[pl.ds(0, CHUNK)], buf, sem).wait()

        def zbody(i, _):
            r = i // (d // 16)
            col = (i % (d // 16)) * 16
            rb0[r, pl.ds(col, 16)] = jnp.zeros((16,), jnp.float32)
            return ()
        lax.fori_loop(0, CHUNK * (d // 16), zbody, ())
        zoffs = [(k * CHUNK, min(CHUNK, per_tile - k * CHUNK))
                 for k in range(-(-per_tile // CHUNK))]
        zd = [
            pltpu.async_copy(
                rb0.at[pl.ds(0, nr)],
                acc_sh.at[pl.ds(s * per_tile + r0, nr)], g0)
            for r0, nr in zoffs
        ]
        pltpu.sync_copy(src_hbm.at[t], srcp)
        pltpu.sync_copy(dst_hbm.at[t, pl.ds(0, GROUP)], dstb.at[0])
        for dsc in zd:
            dsc.wait()
        plsc.subcore_barrier()

        pltpu.async_copy(support_hbm.at[srcp.at[0]], bufs[0], gsem[0])

        def group(gi, _):
            b = gi % 2
            nb = 1 - b
            base = gi * GROUP

            @pl.when(gi + 1 < ngroups)
            def _prefetch():
                off = (gi + 1) * GROUP
                pltpu.async_copy(
                    dst_hbm.at[t, pl.ds(off, GROUP)], dstb.at[nb], isem)

            pend_g = [None, None]
            pend_s = [None, None]
            for jj in range(GROUP):
                r = jj % NBUF
                pr = (jj - 1) % NBUF
                m = base + jj
                if pend_g[r] is None:
                    drain(gsem[r], bufs[r])
                else:
                    pend_g[r].wait()
                pend_s[r] = pltpu.async_copy(
                    bufs[r], acc_sh.at[dstb.at[b, jj]], ssem[r], add=True)
                if pend_s[pr] is not None:
                    pend_s[pr].wait()
                    pend_s[pr] = None
                else:
                    @pl.when(gi > 0)
                    def _d0():
                        drain(ssem[pr], bufs[pr])
                if jj + 1 < GROUP:
                    pend_g[pr] = pltpu.async_copy(
                        support_hbm.at[srcp.at[m + 1]], bufs[pr], gsem[pr])
                else:
                    @pl.when(gi + 1 < ngroups)
                    def _la():
                        pltpu.async_copy(
                            support_hbm.at[srcp.at[m + 1]], bufs[pr], gsem[pr])
            @pl.when(gi + 1 < ngroups)
            def _drain_prefetch():
                pltpu.make_async_copy(
                    dst_hbm.at[t, pl.ds(0, GROUP)], dstb.at[nb], isem).wait()
            return ()
        lax.fori_loop(0, ngroups, group, ())
        drain(ssem[(nch - 1) % NBUF], bufs[(nch - 1) % NBUF])
        plsc.subcore_barrier()

        pend = [None, None]
        for k, (r0, nr) in enumerate(zoffs):
            p = k & 1
            base = s * per_tile + r0
            if pend[p] is not None:
                pend[p].wait()
            pltpu.sync_copy(acc_sh.at[pl.ds(base, nr)], bufs[p].at[pl.ds(0, nr)])
            pend[p] = pltpu.async_copy(
                bufs[p].at[pl.ds(0, nr)], out_hbm.at[c].at[pl.ds(base, nr)],
                gsem[p])
        for p in (0, 1):
            if pend[p] is not None:
                pend[p].wait()

    return sc_scatter


def kernel(x, adj, W1, b1, W2, b2):
    n, d = x.shape
    e = adj.shape[1]
    ch_per_tile = -(-e // (NW * CHUNK))
    ch_per_tile = ((ch_per_tile + GROUP - 1) // GROUP) * GROUP
    e_pad = NW * ch_per_tile * CHUNK

    src = adj[0].astype(jnp.int32)
    dst = adj[1].astype(jnp.int32)
    pad = e_pad - e
    if pad:
        src = jnp.concatenate([src, jnp.zeros((pad,), jnp.int32)])
        dst = jnp.concatenate([dst, jnp.full((pad,), n, jnp.int32)])
    src_t = src.reshape(NW, ch_per_tile, CHUNK)
    dst_t = dst.reshape(NW, ch_per_tile, CHUNK)

    sc_scatter = _make_sc_scatter(n, d, ch_per_tile)

    support1 = _mm_bias(x, W1, b1)
    parts1 = sc_scatter(support1, src_t, dst_t)
    support2 = _combine_relu_mm(parts1, W2, b2, n)
    parts2 = sc_scatter(support2, src_t, dst_t)
    return _combine(parts2, n)


# speedup vs baseline: 1.4222x; 1.4008x over previous
import functools

import jax
import jax.numpy as jnp
from jax import lax
from jax.experimental import pallas as pl
from jax.experimental.pallas import tpu as pltpu
from jax.experimental.pallas import tpu_sc as plsc

NC = 2
NS = 16
NW = NC * NS
CHUNK = 128
NBUF = 2
GROUP = 8
ROW_BLOCK = 1000


def _mm_bias_body(x_ref, w_ref, b_ref, o_ref):
    o_ref[...] = (
        jnp.dot(x_ref[...], w_ref[...], preferred_element_type=jnp.float32)
        + b_ref[...]
    )


def _mm_bias(x, W, b):
    n, d_in = x.shape
    d_out = W.shape[1]
    grid = n // ROW_BLOCK
    return pl.pallas_call(
        _mm_bias_body,
        grid=(grid,),
        in_specs=[
            pl.BlockSpec((ROW_BLOCK, d_in), lambda i: (i, 0)),
            pl.BlockSpec((d_in, d_out), lambda i: (0, 0)),
            pl.BlockSpec((1, d_out), lambda i: (0, 0)),
        ],
        out_specs=pl.BlockSpec((ROW_BLOCK, d_out), lambda i: (i, 0)),
        out_shape=jax.ShapeDtypeStruct((n, d_out), jnp.float32),
    )(x, W, b.reshape(1, d_out))


def _combine_relu_mm_body(acc_ref, w_ref, b_ref, o_ref):
    x1 = jnp.maximum(acc_ref[0] + acc_ref[1], 0.0)
    o_ref[...] = (
        jnp.dot(x1, w_ref[...], preferred_element_type=jnp.float32) + b_ref[...]
    )


def _combine_relu_mm(parts, W, b, n):
    d_in = parts.shape[2]
    d_out = W.shape[1]
    grid = n // ROW_BLOCK
    return pl.pallas_call(
        _combine_relu_mm_body,
        grid=(grid,),
        in_specs=[
            pl.BlockSpec((2, ROW_BLOCK, d_in), lambda i: (0, i, 0)),
            pl.BlockSpec((d_in, d_out), lambda i: (0, 0)),
            pl.BlockSpec((1, d_out), lambda i: (0, 0)),
        ],
        out_specs=pl.BlockSpec((ROW_BLOCK, d_out), lambda i: (i, 0)),
        out_shape=jax.ShapeDtypeStruct((n, d_out), jnp.float32),
    )(parts, W, b.reshape(1, d_out))


def _combine_body(acc_ref, o_ref):
    o_ref[...] = acc_ref[0] + acc_ref[1]


def _combine(parts, n):
    d = parts.shape[2]
    grid = n // ROW_BLOCK
    return pl.pallas_call(
        _combine_body,
        grid=(grid,),
        in_specs=[pl.BlockSpec((2, ROW_BLOCK, d), lambda i: (0, i, 0))],
        out_specs=pl.BlockSpec((ROW_BLOCK, d), lambda i: (i, 0)),
        out_shape=jax.ShapeDtypeStruct((n, d), jnp.float32),
    )(parts)


def _acc_rows(n):
    return ((n + 1 + NS * 8 - 1) // (NS * 8)) * (NS * 8)


def _make_sc_scatter(n, d, ch_per_tile):
    acc_rows = _acc_rows(n)
    per_tile = acc_rows // NS
    mesh = plsc.VectorSubcoreMesh(core_axis_name="c", subcore_axis_name="s")

    @functools.partial(
        pl.kernel,
        out_type=jax.ShapeDtypeStruct((NC, acc_rows, d), jnp.float32),
        mesh=mesh,
        scratch_types=[
            pltpu.VMEM((ch_per_tile, CHUNK), jnp.int32),
            pltpu.VMEM((ch_per_tile, CHUNK), jnp.int32),
            pltpu.VMEM((CHUNK, d), jnp.float32),
            pltpu.VMEM_SHARED((acc_rows, d), jnp.float32),
            pltpu.SemaphoreType.DMA,
        ],
    )
    def sc_scatter(support_hbm, src_hbm, dst_hbm, out_hbm,
                   srcp, dstp, rows_v, acc_sh, sem):
        c = lax.axis_index("c")
        s = lax.axis_index("s")
        t = c * NS + s

        def zbody(i, _):
            r = i // (d // 16)
            col = (i % (d // 16)) * 16
            rows_v[r, pl.ds(col, 16)] = jnp.zeros((16,), jnp.float32)
            return ()
        lax.fori_loop(0, CHUNK * (d // 16), zbody, ())
        zoffs = [(k * CHUNK, min(CHUNK, per_tile - k * CHUNK))
                 for k in range(-(-per_tile // CHUNK))]
        zd = [
            pltpu.async_copy(
                rows_v.at[pl.ds(0, nr)],
                acc_sh.at[pl.ds(s * per_tile + r0, nr)], sem)
            for r0, nr in zoffs
        ]
        pltpu.sync_copy(src_hbm.at[t], srcp)
        pltpu.sync_copy(dst_hbm.at[t], dstp)
        for dsc in zd:
            dsc.wait()
        plsc.subcore_barrier()

        def body(j, _):
            pltpu.async_copy(support_hbm.at[srcp.at[j]], rows_v, sem).wait()
            pltpu.sync_copy(rows_v, acc_sh.at[dstp.at[j]], add=True)
            return ()
        lax.fori_loop(0, ch_per_tile, body, ())
        plsc.subcore_barrier()

        for r0, nr in zoffs:
            base = s * per_tile + r0
            pltpu.sync_copy(acc_sh.at[pl.ds(base, nr)], rows_v.at[pl.ds(0, nr)])
            pltpu.sync_copy(
                rows_v.at[pl.ds(0, nr)], out_hbm.at[c].at[pl.ds(base, nr)])

    return sc_scatter


def kernel(x, adj, W1, b1, W2, b2):
    n, d = x.shape
    e = adj.shape[1]
    ch_per_tile = -(-e // (NW * CHUNK))
    e_pad = NW * ch_per_tile * CHUNK

    src = adj[0].astype(jnp.int32)
    dst = adj[1].astype(jnp.int32)
    pad = e_pad - e
    if pad:
        src = jnp.concatenate([src, jnp.zeros((pad,), jnp.int32)])
        dst = jnp.concatenate([dst, jnp.full((pad,), n, jnp.int32)])
    src_t = src.reshape(NW, ch_per_tile, CHUNK)
    dst_t = dst.reshape(NW, ch_per_tile, CHUNK)

    sc_scatter = _make_sc_scatter(n, d, ch_per_tile)

    support1 = _mm_bias(x, W1, b1)
    parts1 = sc_scatter(support1, src_t, dst_t)
    support2 = _combine_relu_mm(parts1, W2, b2, n)
    parts2 = sc_scatter(support2, src_t, dst_t)
    return _combine(parts2, n)


# speedup vs baseline: 1.4300x; 1.0055x over previous
import functools

import jax
import jax.numpy as jnp
from jax import lax
from jax.experimental import pallas as pl
from jax.experimental.pallas import tpu as pltpu
from jax.experimental.pallas import tpu_sc as plsc

NC = 2
NS = 16
NW = NC * NS
CHUNK = 128
ROW_BLOCK = 1000


def _mm_bias_body(x_ref, w_ref, b_ref, o_ref):
    o_ref[...] = (
        jnp.dot(x_ref[...], w_ref[...], preferred_element_type=jnp.float32)
        + b_ref[...]
    )


def _mm_bias(x, W, b):
    n, d_in = x.shape
    d_out = W.shape[1]
    grid = n // ROW_BLOCK
    return pl.pallas_call(
        _mm_bias_body,
        grid=(grid,),
        in_specs=[
            pl.BlockSpec((ROW_BLOCK, d_in), lambda i: (i, 0)),
            pl.BlockSpec((d_in, d_out), lambda i: (0, 0)),
            pl.BlockSpec((1, d_out), lambda i: (0, 0)),
        ],
        out_specs=pl.BlockSpec((ROW_BLOCK, d_out), lambda i: (i, 0)),
        out_shape=jax.ShapeDtypeStruct((n, d_out), jnp.float32),
    )(x, W, b.reshape(1, d_out))


def _combine_relu_mm_body(acc_ref, w_ref, b_ref, o_ref):
    x1 = jnp.maximum(acc_ref[0] + acc_ref[1], 0.0)
    o_ref[...] = (
        jnp.dot(x1, w_ref[...], preferred_element_type=jnp.float32) + b_ref[...]
    )


def _combine_relu_mm(parts, W, b, n):
    d_in = parts.shape[2]
    d_out = W.shape[1]
    grid = n // ROW_BLOCK
    return pl.pallas_call(
        _combine_relu_mm_body,
        grid=(grid,),
        in_specs=[
            pl.BlockSpec((2, ROW_BLOCK, d_in), lambda i: (0, i, 0)),
            pl.BlockSpec((d_in, d_out), lambda i: (0, 0)),
            pl.BlockSpec((1, d_out), lambda i: (0, 0)),
        ],
        out_specs=pl.BlockSpec((ROW_BLOCK, d_out), lambda i: (i, 0)),
        out_shape=jax.ShapeDtypeStruct((n, d_out), jnp.float32),
    )(parts, W, b.reshape(1, d_out))


def _combine_body(acc_ref, o_ref):
    o_ref[...] = acc_ref[0] + acc_ref[1]


def _combine(parts, n):
    d = parts.shape[2]
    grid = n // ROW_BLOCK
    return pl.pallas_call(
        _combine_body,
        grid=(grid,),
        in_specs=[pl.BlockSpec((2, ROW_BLOCK, d), lambda i: (0, i, 0))],
        out_specs=pl.BlockSpec((ROW_BLOCK, d), lambda i: (i, 0)),
        out_shape=jax.ShapeDtypeStruct((n, d), jnp.float32),
    )(parts)


def _acc_rows(n):
    return ((n + 1 + NS * 8 - 1) // (NS * 8)) * (NS * 8)


def _make_sc_scatter(n, d, ch_per_tile):
    acc_rows = _acc_rows(n)
    per_tile = acc_rows // NS
    mesh = plsc.VectorSubcoreMesh(core_axis_name="c", subcore_axis_name="s")

    @functools.partial(
        pl.kernel,
        out_type=jax.ShapeDtypeStruct((NC, acc_rows, d), jnp.float32),
        mesh=mesh,
        scratch_types=[
            pltpu.VMEM((ch_per_tile, CHUNK), jnp.int32),
            pltpu.VMEM((ch_per_tile, CHUNK), jnp.int32),
            pltpu.VMEM((CHUNK, d), jnp.float32),
            pltpu.VMEM_SHARED((acc_rows, d), jnp.float32),
            pltpu.SemaphoreType.DMA,
        ],
    )
    def sc_scatter(support_hbm, src_hbm, dst_hbm, out_hbm,
                   srcp, dstp, rows_v, acc_sh, sem):
        c = lax.axis_index("c")
        s = lax.axis_index("s")
        t = c * NS + s

        def zbody(i, _):
            r = i // (d // 16)
            col = (i % (d // 16)) * 16
            rows_v[r, pl.ds(col, 16)] = jnp.zeros((16,), jnp.float32)
            return ()
        lax.fori_loop(0, CHUNK * (d // 16), zbody, ())
        zoffs = [(k * CHUNK, min(CHUNK, per_tile - k * CHUNK))
                 for k in range(-(-per_tile // CHUNK))]
        zd = [
            pltpu.async_copy(
                rows_v.at[pl.ds(0, nr)],
                acc_sh.at[pl.ds(s * per_tile + r0, nr)], sem)
            for r0, nr in zoffs
        ]
        pltpu.sync_copy(src_hbm.at[t], srcp)
        pltpu.sync_copy(dst_hbm.at[t], dstp)
        for dsc in zd:
            dsc.wait()
        plsc.subcore_barrier()

        def body(j, _):
            pltpu.async_copy(support_hbm.at[srcp.at[j]], rows_v, sem).wait()
            pltpu.sync_copy(rows_v, acc_sh.at[dstp.at[j]], add=True)
            return ()
        lax.fori_loop(0, ch_per_tile, body, ())
        plsc.subcore_barrier()

        for r0, nr in zoffs:
            base = s * per_tile + r0
            pltpu.sync_copy(acc_sh.at[pl.ds(base, nr)], rows_v.at[pl.ds(0, nr)])
            pltpu.sync_copy(
                rows_v.at[pl.ds(0, nr)], out_hbm.at[c].at[pl.ds(base, nr)])

    return sc_scatter


def kernel(x, adj, W1, b1, W2, b2):
    n, d = x.shape
    e = adj.shape[1]
    ch_per_tile = -(-e // (NW * CHUNK))
    e_pad = NW * ch_per_tile * CHUNK

    src = adj[0].astype(jnp.int32)
    dst = adj[1].astype(jnp.int32)
    pad = e_pad - e
    if pad:
        src = jnp.concatenate([src, jnp.zeros((pad,), jnp.int32)])
        dst = jnp.concatenate([dst, jnp.full((pad,), n, jnp.int32)])
    src_t = src.reshape(NW, ch_per_tile, CHUNK)
    dst_t = dst.reshape(NW, ch_per_tile, CHUNK)

    sc_scatter = _make_sc_scatter(n, d, ch_per_tile)

    support1 = _mm_bias(x, W1, b1)
    parts1 = sc_scatter(support1, src_t, dst_t)
    support2 = _combine_relu_mm(parts1, W2, b2, n)
    parts2 = sc_scatter(support2, src_t, dst_t)
    return _combine(parts2, n)
